# baseline (speedup 1.0000x reference)
import dataclasses

import jax
import jax.numpy as jnp
from jax import lax
from jax.experimental import pallas as pl
from jax.experimental.pallas import tpu as pltpu
from jax.experimental.pallas import tpu_sc as plsc

CONF = 0.3
MAX_WH = 4096.0
MAX_DET = 300
N = 20000
NCLS = 15
B = 4
D = 200
REC = 32
RECROWS = B * N * REC // 128
ROWS_PER_SUB = 1248
ROWS_MAX = 1280
GROUPS = ROWS_MAX // 16
DET_PAD = 304
NCHUNK = DET_PAD // 16
TC_R = 512
BM_LANE = 6 + NCLS
DEG = 0.017453292519943295


def _tc_body(x_ref, rec_ref, bm_ref):
    blk = x_ref[...]
    conf = blk[:, 4:5]
    cls = blk[:, 5:5 + NCLS]
    clsv = cls * conf
    mc = conf > CONF
    validm = jnp.logical_and(mc, clsv > CONF)
    weights = (2 ** jax.lax.iota(jnp.int32, NCLS))[None, :]
    bmv = jnp.sum(validm.astype(jnp.int32) * weights, axis=1)
    bmv = bmv + jnp.where(mc[:, 0], 1 << 15, 0)
    ang = blk[:, 5 + NCLS:D]
    m = jnp.max(ang, axis=1, keepdims=True)
    i180 = lax.broadcasted_iota(jnp.int32, ang.shape, 1)
    aidx = jnp.min(jnp.where(ang == m, i180, jnp.int32(10**9)), axis=1)
    theta = (aidx.astype(jnp.float32) - 90.0) * jnp.float32(DEG)
    rec = jnp.concatenate(
        [blk[:, 0:5], theta[:, None], cls,
         jnp.zeros((TC_R, 128 - 6 - NCLS), jnp.float32)], axis=1)
    rec_ref[...] = rec
    bm_ref[...] = bmv


@jax.jit
def _tc_prepass(x2):
    grid = (B * N + TC_R - 1) // TC_R
    return pl.pallas_call(
        _tc_body,
        grid=(grid,),
        in_specs=[pl.BlockSpec((TC_R, D), lambda i: (i, 0))],
        out_specs=[
            pl.BlockSpec((TC_R, 128), lambda i: (i, 0)),
            pl.BlockSpec((TC_R,), lambda i: (i,)),
        ],
        out_shape=[
            jax.ShapeDtypeStruct((B * N, 128), jnp.float32),
            jax.ShapeDtypeStruct((B * N,), jnp.int32),
        ],
    )(x2)


def _iota():
    return lax.iota(jnp.int32, 16)


def _splat_i32(v):
    return jnp.zeros((16,), jnp.int32) + v


def _cfun(pc_ref, bm_ref, q, t_scalar):
    lo = jnp.zeros((16,), jnp.int32)
    hi = _splat_i32(N)
    nm1 = _splat_i32(N - 1)
    for _ in range(15):
        mid = jnp.minimum(lax.shift_right_logical(lo + hi, 1), nm1)
        pv = plsc.load_gather(pc_ref, [mid])
        cond = pv > q
        hi = jnp.where(cond, mid, hi)
        lo = jnp.where(cond, lo, mid + 1)
    in_t = q < t_scalar
    r = jnp.where(in_t, lo, 0)
    pcm1 = plsc.load_gather(pc_ref, [jnp.maximum(r - 1, 0)])
    pcx = jnp.where(r > 0, pcm1, 0)
    rem = q - pcx
    bmv = plsc.load_gather(bm_ref, [r])
    col = jnp.zeros((16,), jnp.int32)
    seen = jnp.zeros((16,), jnp.int32)
    found = jnp.zeros((16,), jnp.bool_)
    for t in range(NCLS):
        bit = lax.shift_right_logical(bmv, t) & 1
        hit = (bit == 1) & (seen == rem) & jnp.logical_not(found)
        col = jnp.where(hit, t, col)
        found = jnp.logical_or(found, hit)
        seen = seen + bit
    col = jnp.where(in_t, col, 0)
    return r, col


def _sc_body(rec_ref, bm_ref, out_ref, lbm, lpc, lrank, pc_all, bm_all,
             rank_all, idx3, rowbuf, outbuf, c2buf, tot_v, cnt_all,
             spm_pc, spm_bm, spm_rank, spm_cnt, sem):
    c = lax.axis_index("c")
    s = lax.axis_index("s")
    iv = _iota()
    rowbase = ROWS_PER_SUB * s
    nrows = jnp.where(s == 15, ROWS_MAX, ROWS_PER_SUB)

    for bb in range(2):
        b = 2 * c + bb

        pltpu.sync_copy(bm_ref.at[pl.ds(b * N + rowbase, ROWS_MAX)], lbm)

        def p1_group(g, carry):
            vcar, ccar = carry
            rid = iv + 16 * g
            rmask = rid < nrows
            bmv = lbm[pl.ds(16 * g, 16)]
            mc = (lax.shift_right_logical(bmv, 15) == 1) & rmask
            cntv = jnp.zeros((16,), jnp.int32)
            for j in range(NCLS):
                cntv = cntv + (lax.shift_right_logical(bmv, j) & 1)
            cntv = jnp.where(rmask, cntv, 0)
            pcv = plsc.cumsum(cntv) + vcar
            lpc[pl.ds(16 * g, 16)] = pcv
            mci = jnp.where(mc, 1, 0)
            rkv = plsc.cumsum(mci) + ccar
            lrank[pl.ds(16 * g, 16)] = rkv
            return vcar + jnp.sum(cntv), ccar + jnp.sum(mci)

        vc_tot, cc_tot = lax.fori_loop(
            0, GROUPS, p1_group, (jnp.int32(0), jnp.int32(0)))

        totv = jnp.where(iv == 0, vc_tot, jnp.where(iv == 1, cc_tot, 0))
        tot_v[...] = totv
        pltpu.sync_copy(tot_v, spm_cnt.at[s])
        plsc.subcore_barrier()

        pltpu.sync_copy(spm_cnt, cnt_all)
        vc_col = plsc.load_gather(cnt_all, [iv, _splat_i32(0)])
        cc_col = plsc.load_gather(cnt_all, [iv, _splat_i32(1)])
        before = iv < s
        vstart = jnp.sum(jnp.where(before, vc_col, 0))
        cstart = jnp.sum(jnp.where(before, cc_col, 0))
        t_total = jnp.sum(vc_col)

        @pl.loop(0, GROUPS)
        def _adjust(g):
            sl = pl.ds(16 * g, 16)
            lpc[sl] = lpc[sl] + vstart
            lrank[sl] = lrank[sl] + (cstart - 1)

        pltpu.sync_copy(lpc.at[pl.ds(0, ROWS_PER_SUB)],
                        spm_pc.at[pl.ds(rowbase, ROWS_PER_SUB)])
        pltpu.sync_copy(lbm.at[pl.ds(0, ROWS_PER_SUB)],
                        spm_bm.at[pl.ds(rowbase, ROWS_PER_SUB)])
        pltpu.sync_copy(lrank.at[pl.ds(0, ROWS_PER_SUB)],
                        spm_rank.at[pl.ds(rowbase, ROWS_PER_SUB)])

        @pl.when(s == 15)
        def _tail():
            tail = ROWS_MAX - ROWS_PER_SUB
            src = pl.ds(ROWS_PER_SUB, tail)
            dst = pl.ds(16 * ROWS_PER_SUB, tail)
            pltpu.sync_copy(lpc.at[src], spm_pc.at[dst])
            pltpu.sync_copy(lbm.at[src], spm_bm.at[dst])
            pltpu.sync_copy(lrank.at[src], spm_rank.at[dst])

        plsc.subcore_barrier()

        pltpu.sync_copy(spm_pc, pc_all)
        pltpu.sync_copy(spm_bm, bm_all)
        pltpu.sync_copy(spm_rank, rank_all)

        def do_chunk(ch):
            jv = 16 * ch + iv
            r1, _c1 = _cfun(pc_all, bm_all, jv, t_total)
            rk = plsc.load_gather(rank_all, [r1])
            tm1 = jnp.maximum(t_total - 1, 0)
            keep = jnp.clip(rk, 0, tm1)
            r2, c2 = _cfun(pc_all, bm_all, keep, t_total)
            idx3[...] = b * N + r2
            c2buf[...] = c2
            pltpu.async_copy(rec_ref.at[idx3], rowbuf, sem).wait()

            @pl.loop(0, 16)
            def _det(d):
                c2s = plsc.load_gather(c2buf, [_splat_i32(d)])
                hdr = rowbuf[d, pl.ds(0, 16)]
                clsv = rowbuf[d, pl.ds(6, 16)]
                conf_s = plsc.load_gather(
                    rowbuf, [_splat_i32(d), _splat_i32(4)])
                theta = plsc.load_gather(
                    rowbuf, [_splat_i32(d), _splat_i32(5)])
                coff = c2s.astype(jnp.float32) * MAX_WH
                score = jnp.sum(jnp.where(iv == c2s, clsv, 0.0)) * conf_s
                outv = jnp.where(
                    iv < 4, hdr + coff,
                    jnp.where(iv == 4, theta,
                              jnp.where(iv == 5, score,
                                        jnp.where(iv == 6,
                                                  c2s.astype(jnp.float32),
                                                  0.0))))
                live = jnp.where(16 * ch + d < t_total,
                                 jnp.float32(1.0), jnp.float32(0.0))
                outbuf[d, :] = outv * live

            pltpu.sync_copy(outbuf,
                            out_ref.at[b].at[pl.ds(16 * ch, 16), :])

        do_chunk(s)

        @pl.when(s < NCHUNK - 16)
        def _extra():
            do_chunk(16 + s)

        plsc.subcore_barrier()


@jax.jit
def _nms_sc(rec, bm):
    mesh = plsc.VectorSubcoreMesh(core_axis_name="c", subcore_axis_name="s")
    cp = pltpu.CompilerParams()
    fields = pltpu.CompilerParams.__dataclass_fields__
    if "needs_layout_passes" in fields:
        cp = dataclasses.replace(cp, needs_layout_passes=False)
    if "use_tc_tiling_on_sc" in fields:
        cp = dataclasses.replace(cp, use_tc_tiling_on_sc=True)
    kfn = pl.kernel(
        _sc_body,
        out_type=jax.ShapeDtypeStruct((B, DET_PAD, 16), jnp.float32),
        mesh=mesh,
        scratch_types=[
            pltpu.VMEM((ROWS_MAX,), jnp.int32),
            pltpu.VMEM((ROWS_MAX,), jnp.int32),
            pltpu.VMEM((ROWS_MAX,), jnp.int32),
            pltpu.VMEM((N,), jnp.int32),
            pltpu.VMEM((N,), jnp.int32),
            pltpu.VMEM((N,), jnp.int32),
            pltpu.VMEM((16,), jnp.int32),
            pltpu.VMEM((16, 128), jnp.float32),
            pltpu.VMEM((16, 16), jnp.float32),
            pltpu.VMEM((16,), jnp.int32),
            pltpu.VMEM((16,), jnp.int32),
            pltpu.VMEM((16, 16), jnp.int32),
            pltpu.VMEM_SHARED((N,), jnp.int32),
            pltpu.VMEM_SHARED((N,), jnp.int32),
            pltpu.VMEM_SHARED((N,), jnp.int32),
            pltpu.VMEM_SHARED((16, 16), jnp.int32),
            pltpu.SemaphoreType.DMA,
        ],
        compiler_params=cp,
    )
    return kfn(rec, bm)


def kernel(x):
    x2 = x.reshape(B * N, D)
    rec, bm = _tc_prepass(x2)
    outpad = _nms_sc(rec, bm)
    return outpad[:, :MAX_DET, :7]


# speedup vs baseline: 3.3913x; 3.3913x over previous
import dataclasses

import jax
import jax.numpy as jnp
from jax import lax
from jax.experimental import pallas as pl
from jax.experimental.pallas import tpu as pltpu
from jax.experimental.pallas import tpu_sc as plsc

CONF = 0.3
MAX_WH = 4096.0
MAX_DET = 300
N = 20000
NCLS = 15
B = 4
D = 200
REC = 32
RECROWS = B * N * REC // 128
ROWS_PER_SUB = 1248
ROWS_MAX = 1280
GROUPS = ROWS_MAX // 16
DET_PAD = 304
NCHUNK = DET_PAD // 16
TC_R = 512
BM_LANE = 6 + NCLS
DEG = 0.017453292519943295


def _tc_body(x_ref, rec_ref, bm_ref):
    blk = x_ref[0]
    conf = blk[4, :]
    mc = conf > CONF
    bmv = jnp.where(mc, 1 << 15, 0)
    for j in range(NCLS):
        vj = jnp.logical_and(mc, blk[5 + j, :] * conf > CONF)
        bmv = bmv + jnp.where(vj, 1 << j, 0)
    ang = blk[5 + NCLS:D, :]
    m = jnp.max(ang, axis=0, keepdims=True)
    i180 = lax.broadcasted_iota(jnp.int32, ang.shape, 0)
    aidx = jnp.min(jnp.where(ang == m, i180, jnp.int32(10**9)), axis=0)
    theta = (aidx.astype(jnp.float32) - 90.0) * jnp.float32(DEG)
    rect = jnp.concatenate([blk[0:5, :], theta[None, :], blk[5:5 + NCLS, :]],
                           axis=0)
    rec = jnp.transpose(rect)
    rec_ref[0] = jnp.concatenate(
        [rec, jnp.zeros((TC_R, 128 - 6 - NCLS), jnp.float32)], axis=1)
    bm_ref[0, 0] = bmv


@jax.jit
def _tc_prepass(xt):
    grid = (N + TC_R - 1) // TC_R
    return pl.pallas_call(
        _tc_body,
        grid=(B, grid),
        in_specs=[pl.BlockSpec((1, D, TC_R), lambda b, i: (b, 0, i))],
        out_specs=[
            pl.BlockSpec((1, TC_R, 128), lambda b, i: (b, i, 0)),
            pl.BlockSpec((1, 1, TC_R), lambda b, i: (b, 0, i)),
        ],
        out_shape=[
            jax.ShapeDtypeStruct((B, N, 128), jnp.float32),
            jax.ShapeDtypeStruct((B, 1, N), jnp.int32),
        ],
    )(xt)


def _iota():
    return lax.iota(jnp.int32, 16)


def _splat_i32(v):
    return jnp.zeros((16,), jnp.int32) + v


def _cfun(pc_ref, bm_ref, q, t_scalar):
    lo = jnp.zeros((16,), jnp.int32)
    hi = _splat_i32(N)
    nm1 = _splat_i32(N - 1)
    for _ in range(15):
        mid = jnp.minimum(lax.shift_right_logical(lo + hi, 1), nm1)
        pv = plsc.load_gather(pc_ref, [mid])
        cond = pv > q
        hi = jnp.where(cond, mid, hi)
        lo = jnp.where(cond, lo, mid + 1)
    in_t = q < t_scalar
    r = jnp.where(in_t, lo, 0)
    pcm1 = plsc.load_gather(pc_ref, [jnp.maximum(r - 1, 0)])
    pcx = jnp.where(r > 0, pcm1, 0)
    rem = q - pcx
    bmv = plsc.load_gather(bm_ref, [r])
    col = jnp.zeros((16,), jnp.int32)
    seen = jnp.zeros((16,), jnp.int32)
    found = jnp.zeros((16,), jnp.bool_)
    for t in range(NCLS):
        bit = lax.shift_right_logical(bmv, t) & 1
        hit = (bit == 1) & (seen == rem) & jnp.logical_not(found)
        col = jnp.where(hit, t, col)
        found = jnp.logical_or(found, hit)
        seen = seen + bit
    col = jnp.where(in_t, col, 0)
    return r, col


def _sc_body(rec_ref, bm_ref, out_ref, lbm, lpc, lrank, pc_all, bm_all,
             rank_all, idx3, rowbuf, outbuf, c2buf, tot_v, cnt_all,
             spm_pc, spm_bm, spm_rank, spm_cnt, sem):
    c = lax.axis_index("c")
    s = lax.axis_index("s")
    iv = _iota()
    rowbase = ROWS_PER_SUB * s
    nrows = jnp.where(s == 15, ROWS_MAX, ROWS_PER_SUB)

    for bb in range(2):
        b = 2 * c + bb

        pltpu.sync_copy(bm_ref.at[b].at[0].at[pl.ds(rowbase, ROWS_MAX)], lbm)

        def p1_group(g, carry):
            vcar, ccar = carry
            rid = iv + 16 * g
            rmask = rid < nrows
            bmv = lbm[pl.ds(16 * g, 16)]
            mc = (lax.shift_right_logical(bmv, 15) == 1) & rmask
            cntv = jnp.zeros((16,), jnp.int32)
            for j in range(NCLS):
                cntv = cntv + (lax.shift_right_logical(bmv, j) & 1)
            cntv = jnp.where(rmask, cntv, 0)
            pcv = plsc.cumsum(cntv) + vcar
            lpc[pl.ds(16 * g, 16)] = pcv
            mci = jnp.where(mc, 1, 0)
            rkv = plsc.cumsum(mci) + ccar
            lrank[pl.ds(16 * g, 16)] = rkv
            return vcar + jnp.sum(cntv), ccar + jnp.sum(mci)

        vc_tot, cc_tot = lax.fori_loop(
            0, GROUPS, p1_group, (jnp.int32(0), jnp.int32(0)))

        totv = jnp.where(iv == 0, vc_tot, jnp.where(iv == 1, cc_tot, 0))
        tot_v[...] = totv
        pltpu.sync_copy(tot_v, spm_cnt.at[s])
        plsc.subcore_barrier()

        pltpu.sync_copy(spm_cnt, cnt_all)
        vc_col = plsc.load_gather(cnt_all, [iv, _splat_i32(0)])
        cc_col = plsc.load_gather(cnt_all, [iv, _splat_i32(1)])
        before = iv < s
        vstart = jnp.sum(jnp.where(before, vc_col, 0))
        cstart = jnp.sum(jnp.where(before, cc_col, 0))
        t_total = jnp.sum(vc_col)

        @pl.loop(0, GROUPS)
        def _adjust(g):
            sl = pl.ds(16 * g, 16)
            lpc[sl] = lpc[sl] + vstart
            lrank[sl] = lrank[sl] + (cstart - 1)

        pltpu.sync_copy(lpc.at[pl.ds(0, ROWS_PER_SUB)],
                        spm_pc.at[pl.ds(rowbase, ROWS_PER_SUB)])
        pltpu.sync_copy(lbm.at[pl.ds(0, ROWS_PER_SUB)],
                        spm_bm.at[pl.ds(rowbase, ROWS_PER_SUB)])
        pltpu.sync_copy(lrank.at[pl.ds(0, ROWS_PER_SUB)],
                        spm_rank.at[pl.ds(rowbase, ROWS_PER_SUB)])

        @pl.when(s == 15)
        def _tail():
            tail = ROWS_MAX - ROWS_PER_SUB
            src = pl.ds(ROWS_PER_SUB, tail)
            dst = pl.ds(16 * ROWS_PER_SUB, tail)
            pltpu.sync_copy(lpc.at[src], spm_pc.at[dst])
            pltpu.sync_copy(lbm.at[src], spm_bm.at[dst])
            pltpu.sync_copy(lrank.at[src], spm_rank.at[dst])

        plsc.subcore_barrier()

        pltpu.sync_copy(spm_pc, pc_all)
        pltpu.sync_copy(spm_bm, bm_all)
        pltpu.sync_copy(spm_rank, rank_all)

        def do_chunk(ch):
            jv = 16 * ch + iv
            r1, _c1 = _cfun(pc_all, bm_all, jv, t_total)
            rk = plsc.load_gather(rank_all, [r1])
            tm1 = jnp.maximum(t_total - 1, 0)
            keep = jnp.clip(rk, 0, tm1)
            r2, c2 = _cfun(pc_all, bm_all, keep, t_total)
            idx3[...] = b * N + r2
            c2buf[...] = c2
            pltpu.async_copy(rec_ref.at[idx3], rowbuf, sem).wait()

            @pl.loop(0, 16)
            def _det(d):
                c2s = plsc.load_gather(c2buf, [_splat_i32(d)])
                hdr = rowbuf[d, pl.ds(0, 16)]
                clsv = rowbuf[d, pl.ds(6, 16)]
                conf_s = plsc.load_gather(
                    rowbuf, [_splat_i32(d), _splat_i32(4)])
                theta = plsc.load_gather(
                    rowbuf, [_splat_i32(d), _splat_i32(5)])
                coff = c2s.astype(jnp.float32) * MAX_WH
                score = jnp.sum(jnp.where(iv == c2s, clsv, 0.0)) * conf_s
                outv = jnp.where(
                    iv < 4, hdr + coff,
                    jnp.where(iv == 4, theta,
                              jnp.where(iv == 5, score,
                                        jnp.where(iv == 6,
                                                  c2s.astype(jnp.float32),
                                                  0.0))))
                live = jnp.where(16 * ch + d < t_total,
                                 jnp.float32(1.0), jnp.float32(0.0))
                outbuf[d, :] = outv * live

            pltpu.sync_copy(outbuf,
                            out_ref.at[b].at[pl.ds(16 * ch, 16), :])

        do_chunk(s)

        @pl.when(s < NCHUNK - 16)
        def _extra():
            do_chunk(16 + s)

        plsc.subcore_barrier()


@jax.jit
def _nms_sc(rec, bm):
    mesh = plsc.VectorSubcoreMesh(core_axis_name="c", subcore_axis_name="s")
    cp = pltpu.CompilerParams()
    fields = pltpu.CompilerParams.__dataclass_fields__
    if "needs_layout_passes" in fields:
        cp = dataclasses.replace(cp, needs_layout_passes=False)
    if "use_tc_tiling_on_sc" in fields:
        cp = dataclasses.replace(cp, use_tc_tiling_on_sc=True)
    kfn = pl.kernel(
        _sc_body,
        out_type=jax.ShapeDtypeStruct((B, DET_PAD, 16), jnp.float32),
        mesh=mesh,
        scratch_types=[
            pltpu.VMEM((ROWS_MAX,), jnp.int32),
            pltpu.VMEM((ROWS_MAX,), jnp.int32),
            pltpu.VMEM((ROWS_MAX,), jnp.int32),
            pltpu.VMEM((N,), jnp.int32),
            pltpu.VMEM((N,), jnp.int32),
            pltpu.VMEM((N,), jnp.int32),
            pltpu.VMEM((16,), jnp.int32),
            pltpu.VMEM((16, 128), jnp.float32),
            pltpu.VMEM((16, 16), jnp.float32),
            pltpu.VMEM((16,), jnp.int32),
            pltpu.VMEM((16,), jnp.int32),
            pltpu.VMEM((16, 16), jnp.int32),
            pltpu.VMEM_SHARED((N,), jnp.int32),
            pltpu.VMEM_SHARED((N,), jnp.int32),
            pltpu.VMEM_SHARED((N,), jnp.int32),
            pltpu.VMEM_SHARED((16, 16), jnp.int32),
            pltpu.SemaphoreType.DMA,
        ],
        compiler_params=cp,
    )
    return kfn(rec, bm)


def kernel(x):
    xt = x.reshape(B, N, D).transpose(0, 2, 1)
    rec, bm = _tc_prepass(xt)
    outpad = _nms_sc(rec.reshape(B * N, 128), bm)
    return outpad[:, :MAX_DET, :7]


# speedup vs baseline: 5.4341x; 1.6024x over previous
import dataclasses

import jax
import jax.numpy as jnp
from jax import lax
from jax.experimental import pallas as pl
from jax.experimental.pallas import tpu as pltpu
from jax.experimental.pallas import tpu_sc as plsc

CONF = 0.3
MAX_WH = 4096.0
MAX_DET = 300
N = 20000
NCLS = 15
B = 4
D = 200
REC = 32
RECROWS = B * N * REC // 128
ROWS_PER_SUB = 1248
ROWS_MAX = 1280
GROUPS = ROWS_MAX // 16
DET_PAD = 304
NCHUNK = DET_PAD // 16
TC_R = 2048
BM_LANE = 6 + NCLS
DEG = 0.017453292519943295


def _tc_body(x_ref, rec_ref, bm_ref):
    blk = x_ref[0]
    conf = blk[4, :]
    mc = conf > CONF
    bmv = jnp.where(mc, 1 << 15, 0)
    for j in range(NCLS):
        vj = jnp.logical_and(mc, blk[5 + j, :] * conf > CONF)
        bmv = bmv + jnp.where(vj, 1 << j, 0)
    ang = blk[5 + NCLS:D, :]
    m = jnp.max(ang, axis=0, keepdims=True)
    i180 = lax.broadcasted_iota(jnp.int32, ang.shape, 0)
    aidx = jnp.min(jnp.where(ang == m, i180, jnp.int32(10**9)), axis=0)
    theta = (aidx.astype(jnp.float32) - 90.0) * jnp.float32(DEG)
    rect = jnp.concatenate([blk[0:5, :], theta[None, :], blk[5:5 + NCLS, :]],
                           axis=0)
    rec = jnp.transpose(rect)
    rec_ref[0] = jnp.concatenate(
        [rec, jnp.zeros((TC_R, 128 - 6 - NCLS), jnp.float32)], axis=1)
    bm_ref[0, 0] = bmv


@jax.jit
def _tc_prepass(xt):
    grid = (N + TC_R - 1) // TC_R
    return pl.pallas_call(
        _tc_body,
        grid=(B, grid),
        in_specs=[pl.BlockSpec((1, D, TC_R), lambda b, i: (b, 0, i))],
        out_specs=[
            pl.BlockSpec((1, TC_R, 128), lambda b, i: (b, i, 0)),
            pl.BlockSpec((1, 1, TC_R), lambda b, i: (b, 0, i)),
        ],
        out_shape=[
            jax.ShapeDtypeStruct((B, N, 128), jnp.float32),
            jax.ShapeDtypeStruct((B, 1, N), jnp.int32),
        ],
        compiler_params=pltpu.CompilerParams(
            dimension_semantics=("parallel", "parallel")),
    )(xt)


def _iota():
    return lax.iota(jnp.int32, 16)


def _splat_i32(v):
    return jnp.zeros((16,), jnp.int32) + v


def _cfun(pc_ref, bm_ref, q, t_scalar):
    lo = jnp.zeros((16,), jnp.int32)
    hi = _splat_i32(N)
    nm1 = _splat_i32(N - 1)
    for _ in range(15):
        mid = jnp.minimum(lax.shift_right_logical(lo + hi, 1), nm1)
        pv = plsc.load_gather(pc_ref, [mid])
        cond = pv > q
        hi = jnp.where(cond, mid, hi)
        lo = jnp.where(cond, lo, mid + 1)
    in_t = q < t_scalar
    r = jnp.where(in_t, lo, 0)
    pcm1 = plsc.load_gather(pc_ref, [jnp.maximum(r - 1, 0)])
    pcx = jnp.where(r > 0, pcm1, 0)
    rem = q - pcx
    bmv = plsc.load_gather(bm_ref, [r])
    col = jnp.zeros((16,), jnp.int32)
    seen = jnp.zeros((16,), jnp.int32)
    found = jnp.zeros((16,), jnp.bool_)
    for t in range(NCLS):
        bit = lax.shift_right_logical(bmv, t) & 1
        hit = (bit == 1) & (seen == rem) & jnp.logical_not(found)
        col = jnp.where(hit, t, col)
        found = jnp.logical_or(found, hit)
        seen = seen + bit
    col = jnp.where(in_t, col, 0)
    return r, col


def _sc_body(rec_ref, bm_ref, out_ref, lbm, lpc, lrank, pc_all, bm_all,
             rank_all, idx3, rowbuf, outbuf, c2buf, tot_v, cnt_all,
             spm_pc, spm_bm, spm_rank, spm_cnt, sem):
    c = lax.axis_index("c")
    s = lax.axis_index("s")
    iv = _iota()
    rowbase = ROWS_PER_SUB * s
    nrows = jnp.where(s == 15, ROWS_MAX, ROWS_PER_SUB)

    for bb in range(2):
        b = 2 * c + bb

        pltpu.sync_copy(bm_ref.at[b].at[0].at[pl.ds(rowbase, ROWS_MAX)], lbm)

        def p1_group(g, carry):
            vcar, ccar = carry
            rid = iv + 16 * g
            rmask = rid < nrows
            bmv = lbm[pl.ds(16 * g, 16)]
            mc = (lax.shift_right_logical(bmv, 15) == 1) & rmask
            cntv = jnp.zeros((16,), jnp.int32)
            for j in range(NCLS):
                cntv = cntv + (lax.shift_right_logical(bmv, j) & 1)
            cntv = jnp.where(rmask, cntv, 0)
            pcv = plsc.cumsum(cntv) + vcar
            lpc[pl.ds(16 * g, 16)] = pcv
            mci = jnp.where(mc, 1, 0)
            rkv = plsc.cumsum(mci) + ccar
            lrank[pl.ds(16 * g, 16)] = rkv
            return vcar + jnp.sum(cntv), ccar + jnp.sum(mci)

        vc_tot, cc_tot = lax.fori_loop(
            0, GROUPS, p1_group, (jnp.int32(0), jnp.int32(0)))

        totv = jnp.where(iv == 0, vc_tot, jnp.where(iv == 1, cc_tot, 0))
        tot_v[...] = totv
        pltpu.sync_copy(tot_v, spm_cnt.at[s])
        plsc.subcore_barrier()

        pltpu.sync_copy(spm_cnt, cnt_all)
        vc_col = plsc.load_gather(cnt_all, [iv, _splat_i32(0)])
        cc_col = plsc.load_gather(cnt_all, [iv, _splat_i32(1)])
        before = iv < s
        vstart = jnp.sum(jnp.where(before, vc_col, 0))
        cstart = jnp.sum(jnp.where(before, cc_col, 0))
        t_total = jnp.sum(vc_col)

        @pl.loop(0, GROUPS)
        def _adjust(g):
            sl = pl.ds(16 * g, 16)
            lpc[sl] = lpc[sl] + vstart
            lrank[sl] = lrank[sl] + (cstart - 1)

        pltpu.sync_copy(lpc.at[pl.ds(0, ROWS_PER_SUB)],
                        spm_pc.at[pl.ds(rowbase, ROWS_PER_SUB)])
        pltpu.sync_copy(lbm.at[pl.ds(0, ROWS_PER_SUB)],
                        spm_bm.at[pl.ds(rowbase, ROWS_PER_SUB)])
        pltpu.sync_copy(lrank.at[pl.ds(0, ROWS_PER_SUB)],
                        spm_rank.at[pl.ds(rowbase, ROWS_PER_SUB)])

        @pl.when(s == 15)
        def _tail():
            tail = ROWS_MAX - ROWS_PER_SUB
            src = pl.ds(ROWS_PER_SUB, tail)
            dst = pl.ds(16 * ROWS_PER_SUB, tail)
            pltpu.sync_copy(lpc.at[src], spm_pc.at[dst])
            pltpu.sync_copy(lbm.at[src], spm_bm.at[dst])
            pltpu.sync_copy(lrank.at[src], spm_rank.at[dst])

        plsc.subcore_barrier()

        pltpu.sync_copy(spm_pc, pc_all)
        pltpu.sync_copy(spm_bm, bm_all)
        pltpu.sync_copy(spm_rank, rank_all)

        def do_chunk(ch):
            jv = 16 * ch + iv
            r1, _c1 = _cfun(pc_all, bm_all, jv, t_total)
            rk = plsc.load_gather(rank_all, [r1])
            tm1 = jnp.maximum(t_total - 1, 0)
            keep = jnp.clip(rk, 0, tm1)
            r2, c2 = _cfun(pc_all, bm_all, keep, t_total)
            idx3[...] = b * N + r2
            c2buf[...] = c2
            pltpu.async_copy(rec_ref.at[idx3], rowbuf, sem).wait()

            @pl.loop(0, 16)
            def _det(d):
                c2s = plsc.load_gather(c2buf, [_splat_i32(d)])
                hdr = rowbuf[d, pl.ds(0, 16)]
                clsv = rowbuf[d, pl.ds(6, 16)]
                conf_s = plsc.load_gather(
                    rowbuf, [_splat_i32(d), _splat_i32(4)])
                theta = plsc.load_gather(
                    rowbuf, [_splat_i32(d), _splat_i32(5)])
                coff = c2s.astype(jnp.float32) * MAX_WH
                score = jnp.sum(jnp.where(iv == c2s, clsv, 0.0)) * conf_s
                outv = jnp.where(
                    iv < 4, hdr + coff,
                    jnp.where(iv == 4, theta,
                              jnp.where(iv == 5, score,
                                        jnp.where(iv == 6,
                                                  c2s.astype(jnp.float32),
                                                  0.0))))
                live = jnp.where(16 * ch + d < t_total,
                                 jnp.float32(1.0), jnp.float32(0.0))
                outbuf[d, :] = outv * live

            pltpu.sync_copy(outbuf,
                            out_ref.at[b].at[pl.ds(16 * ch, 16), :])

        do_chunk(s)

        @pl.when(s < NCHUNK - 16)
        def _extra():
            do_chunk(16 + s)

        plsc.subcore_barrier()


@jax.jit
def _nms_sc(rec, bm):
    mesh = plsc.VectorSubcoreMesh(core_axis_name="c", subcore_axis_name="s")
    cp = pltpu.CompilerParams()
    fields = pltpu.CompilerParams.__dataclass_fields__
    if "needs_layout_passes" in fields:
        cp = dataclasses.replace(cp, needs_layout_passes=False)
    if "use_tc_tiling_on_sc" in fields:
        cp = dataclasses.replace(cp, use_tc_tiling_on_sc=True)
    kfn = pl.kernel(
        _sc_body,
        out_type=jax.ShapeDtypeStruct((B, DET_PAD, 16), jnp.float32),
        mesh=mesh,
        scratch_types=[
            pltpu.VMEM((ROWS_MAX,), jnp.int32),
            pltpu.VMEM((ROWS_MAX,), jnp.int32),
            pltpu.VMEM((ROWS_MAX,), jnp.int32),
            pltpu.VMEM((N,), jnp.int32),
            pltpu.VMEM((N,), jnp.int32),
            pltpu.VMEM((N,), jnp.int32),
            pltpu.VMEM((16,), jnp.int32),
            pltpu.VMEM((16, 128), jnp.float32),
            pltpu.VMEM((16, 16), jnp.float32),
            pltpu.VMEM((16,), jnp.int32),
            pltpu.VMEM((16,), jnp.int32),
            pltpu.VMEM((16, 16), jnp.int32),
            pltpu.VMEM_SHARED((N,), jnp.int32),
            pltpu.VMEM_SHARED((N,), jnp.int32),
            pltpu.VMEM_SHARED((N,), jnp.int32),
            pltpu.VMEM_SHARED((16, 16), jnp.int32),
            pltpu.SemaphoreType.DMA,
        ],
        compiler_params=cp,
    )
    return kfn(rec, bm)


def kernel(x):
    xt = x.reshape(B, N, D).transpose(0, 2, 1)
    rec, bm = _tc_prepass(xt)
    outpad = _nms_sc(rec.reshape(B * N, 128), bm)
    return outpad[:, :MAX_DET, :7]


# speedup vs baseline: 6.3158x; 1.1623x over previous
import dataclasses

import jax
import jax.numpy as jnp
from jax import lax
from jax.experimental import pallas as pl
from jax.experimental.pallas import tpu as pltpu
from jax.experimental.pallas import tpu_sc as plsc

CONF = 0.3
MAX_WH = 4096.0
MAX_DET = 300
N = 20000
NCLS = 15
B = 4
D = 200
REC = 32
RECROWS = B * N * REC // 128
ROWS_PER_SUB = 1248
ROWS_MAX = 1280
GROUPS = ROWS_MAX // 16
DET_PAD = 304
NCHUNK = DET_PAD // 16
TC_R = 5120
BM_LANE = 6 + NCLS
DEG = 0.017453292519943295


def _tc_body(x_ref, rec_ref, bm_ref):
    blk = x_ref[0]
    conf = blk[4, :]
    mc = conf > CONF
    bmv = jnp.where(mc, 1 << 15, 0)
    for j in range(NCLS):
        vj = jnp.logical_and(mc, blk[5 + j, :] * conf > CONF)
        bmv = bmv + jnp.where(vj, 1 << j, 0)
    ang = blk[5 + NCLS:D, :]
    m = jnp.max(ang, axis=0, keepdims=True)
    i180 = lax.broadcasted_iota(jnp.int32, ang.shape, 0)
    aidx = jnp.min(jnp.where(ang == m, i180, jnp.int32(10**9)), axis=0)
    theta = (aidx.astype(jnp.float32) - 90.0) * jnp.float32(DEG)
    rect = jnp.concatenate([blk[0:5, :], theta[None, :], blk[5:5 + NCLS, :]],
                           axis=0)
    rec = jnp.transpose(rect)
    rec_ref[0] = jnp.concatenate(
        [rec, jnp.zeros((TC_R, 128 - 6 - NCLS), jnp.float32)], axis=1)
    bm_ref[0, 0] = bmv


@jax.jit
def _tc_prepass(xt):
    grid = (N + TC_R - 1) // TC_R
    return pl.pallas_call(
        _tc_body,
        grid=(B, grid),
        in_specs=[pl.BlockSpec((1, D, TC_R), lambda b, i: (b, 0, i))],
        out_specs=[
            pl.BlockSpec((1, TC_R, 128), lambda b, i: (b, i, 0)),
            pl.BlockSpec((1, 1, TC_R), lambda b, i: (b, 0, i)),
        ],
        out_shape=[
            jax.ShapeDtypeStruct((B, N, 128), jnp.float32),
            jax.ShapeDtypeStruct((B, 1, N), jnp.int32),
        ],
        compiler_params=pltpu.CompilerParams(
            dimension_semantics=("parallel", "parallel")),
    )(xt)


def _iota():
    return lax.iota(jnp.int32, 16)


def _splat_i32(v):
    return jnp.zeros((16,), jnp.int32) + v


def _cfun(pc_ref, bm_ref, q, t_scalar):
    lo = jnp.zeros((16,), jnp.int32)
    hi = _splat_i32(N)
    nm1 = _splat_i32(N - 1)
    for _ in range(15):
        mid = jnp.minimum(lax.shift_right_logical(lo + hi, 1), nm1)
        pv = plsc.load_gather(pc_ref, [mid])
        cond = pv > q
        hi = jnp.where(cond, mid, hi)
        lo = jnp.where(cond, lo, mid + 1)
    in_t = q < t_scalar
    r = jnp.where(in_t, lo, 0)
    pcm1 = plsc.load_gather(pc_ref, [jnp.maximum(r - 1, 0)])
    pcx = jnp.where(r > 0, pcm1, 0)
    rem = q - pcx
    bmv = plsc.load_gather(bm_ref, [r])
    col = jnp.zeros((16,), jnp.int32)
    seen = jnp.zeros((16,), jnp.int32)
    found = jnp.zeros((16,), jnp.bool_)
    for t in range(NCLS):
        bit = lax.shift_right_logical(bmv, t) & 1
        hit = (bit == 1) & (seen == rem) & jnp.logical_not(found)
        col = jnp.where(hit, t, col)
        found = jnp.logical_or(found, hit)
        seen = seen + bit
    col = jnp.where(in_t, col, 0)
    return r, col


def _sc_body(rec_ref, bm_ref, out_ref, lbm, lpc, lrank, pc_all, bm_all,
             rank_all, idx3, rowbuf, outbuf, c2buf, tot_v, cnt_all,
             spm_pc, spm_bm, spm_rank, spm_cnt, sem):
    c = lax.axis_index("c")
    s = lax.axis_index("s")
    iv = _iota()
    rowbase = ROWS_PER_SUB * s
    nrows = jnp.where(s == 15, ROWS_MAX, ROWS_PER_SUB)

    for bb in range(2):
        b = 2 * c + bb

        pltpu.sync_copy(bm_ref.at[b].at[0].at[pl.ds(rowbase, ROWS_MAX)], lbm)

        def p1_group(g, carry):
            vcar, ccar = carry
            rid = iv + 16 * g
            rmask = rid < nrows
            bmv = lbm[pl.ds(16 * g, 16)]
            mc = (lax.shift_right_logical(bmv, 15) == 1) & rmask
            cntv = jnp.zeros((16,), jnp.int32)
            for j in range(NCLS):
                cntv = cntv + (lax.shift_right_logical(bmv, j) & 1)
            cntv = jnp.where(rmask, cntv, 0)
            pcv = plsc.cumsum(cntv) + vcar
            lpc[pl.ds(16 * g, 16)] = pcv
            mci = jnp.where(mc, 1, 0)
            rkv = plsc.cumsum(mci) + ccar
            lrank[pl.ds(16 * g, 16)] = rkv
            return vcar + jnp.sum(cntv), ccar + jnp.sum(mci)

        vc_tot, cc_tot = lax.fori_loop(
            0, GROUPS, p1_group, (jnp.int32(0), jnp.int32(0)))

        totv = jnp.where(iv == 0, vc_tot, jnp.where(iv == 1, cc_tot, 0))
        tot_v[...] = totv
        pltpu.sync_copy(tot_v, spm_cnt.at[s])
        plsc.subcore_barrier()

        pltpu.sync_copy(spm_cnt, cnt_all)
        vc_col = plsc.load_gather(cnt_all, [iv, _splat_i32(0)])
        cc_col = plsc.load_gather(cnt_all, [iv, _splat_i32(1)])
        before = iv < s
        vstart = jnp.sum(jnp.where(before, vc_col, 0))
        cstart = jnp.sum(jnp.where(before, cc_col, 0))
        t_total = jnp.sum(vc_col)

        @pl.loop(0, GROUPS)
        def _adjust(g):
            sl = pl.ds(16 * g, 16)
            lpc[sl] = lpc[sl] + vstart
            lrank[sl] = lrank[sl] + (cstart - 1)

        pltpu.sync_copy(lpc.at[pl.ds(0, ROWS_PER_SUB)],
                        spm_pc.at[pl.ds(rowbase, ROWS_PER_SUB)])
        pltpu.sync_copy(lbm.at[pl.ds(0, ROWS_PER_SUB)],
                        spm_bm.at[pl.ds(rowbase, ROWS_PER_SUB)])
        pltpu.sync_copy(lrank.at[pl.ds(0, ROWS_PER_SUB)],
                        spm_rank.at[pl.ds(rowbase, ROWS_PER_SUB)])

        @pl.when(s == 15)
        def _tail():
            tail = ROWS_MAX - ROWS_PER_SUB
            src = pl.ds(ROWS_PER_SUB, tail)
            dst = pl.ds(16 * ROWS_PER_SUB, tail)
            pltpu.sync_copy(lpc.at[src], spm_pc.at[dst])
            pltpu.sync_copy(lbm.at[src], spm_bm.at[dst])
            pltpu.sync_copy(lrank.at[src], spm_rank.at[dst])

        plsc.subcore_barrier()

        pltpu.sync_copy(spm_pc, pc_all)
        pltpu.sync_copy(spm_bm, bm_all)
        pltpu.sync_copy(spm_rank, rank_all)

        def do_chunk(ch):
            jv = 16 * ch + iv
            r1, _c1 = _cfun(pc_all, bm_all, jv, t_total)
            rk = plsc.load_gather(rank_all, [r1])
            tm1 = jnp.maximum(t_total - 1, 0)
            keep = jnp.clip(rk, 0, tm1)
            r2, c2 = _cfun(pc_all, bm_all, keep, t_total)
            idx3[...] = b * N + r2
            c2buf[...] = c2
            pltpu.async_copy(rec_ref.at[idx3], rowbuf, sem).wait()

            @pl.loop(0, 16)
            def _det(d):
                c2s = plsc.load_gather(c2buf, [_splat_i32(d)])
                hdr = rowbuf[d, pl.ds(0, 16)]
                clsv = rowbuf[d, pl.ds(6, 16)]
                conf_s = plsc.load_gather(
                    rowbuf, [_splat_i32(d), _splat_i32(4)])
                theta = plsc.load_gather(
                    rowbuf, [_splat_i32(d), _splat_i32(5)])
                coff = c2s.astype(jnp.float32) * MAX_WH
                score = jnp.sum(jnp.where(iv == c2s, clsv, 0.0)) * conf_s
                outv = jnp.where(
                    iv < 4, hdr + coff,
                    jnp.where(iv == 4, theta,
                              jnp.where(iv == 5, score,
                                        jnp.where(iv == 6,
                                                  c2s.astype(jnp.float32),
                                                  0.0))))
                live = jnp.where(16 * ch + d < t_total,
                                 jnp.float32(1.0), jnp.float32(0.0))
                outbuf[d, :] = outv * live

            pltpu.sync_copy(outbuf,
                            out_ref.at[b].at[pl.ds(16 * ch, 16), :])

        do_chunk(s)

        @pl.when(s < NCHUNK - 16)
        def _extra():
            do_chunk(16 + s)

        plsc.subcore_barrier()


@jax.jit
def _nms_sc(rec, bm):
    mesh = plsc.VectorSubcoreMesh(core_axis_name="c", subcore_axis_name="s")
    cp = pltpu.CompilerParams()
    fields = pltpu.CompilerParams.__dataclass_fields__
    if "needs_layout_passes" in fields:
        cp = dataclasses.replace(cp, needs_layout_passes=False)
    if "use_tc_tiling_on_sc" in fields:
        cp = dataclasses.replace(cp, use_tc_tiling_on_sc=True)
    kfn = pl.kernel(
        _sc_body,
        out_type=jax.ShapeDtypeStruct((B, DET_PAD, 16), jnp.float32),
        mesh=mesh,
        scratch_types=[
            pltpu.VMEM((ROWS_MAX,), jnp.int32),
            pltpu.VMEM((ROWS_MAX,), jnp.int32),
            pltpu.VMEM((ROWS_MAX,), jnp.int32),
            pltpu.VMEM((N,), jnp.int32),
            pltpu.VMEM((N,), jnp.int32),
            pltpu.VMEM((N,), jnp.int32),
            pltpu.VMEM((16,), jnp.int32),
            pltpu.VMEM((16, 128), jnp.float32),
            pltpu.VMEM((16, 16), jnp.float32),
            pltpu.VMEM((16,), jnp.int32),
            pltpu.VMEM((16,), jnp.int32),
            pltpu.VMEM((16, 16), jnp.int32),
            pltpu.VMEM_SHARED((N,), jnp.int32),
            pltpu.VMEM_SHARED((N,), jnp.int32),
            pltpu.VMEM_SHARED((N,), jnp.int32),
            pltpu.VMEM_SHARED((16, 16), jnp.int32),
            pltpu.SemaphoreType.DMA,
        ],
        compiler_params=cp,
    )
    return kfn(rec, bm)


def kernel(x):
    xt = x.reshape(B, N, D).transpose(0, 2, 1)
    rec, bm = _tc_prepass(xt)
    outpad = _nms_sc(rec.reshape(B * N, 128), bm)
    return outpad[:, :MAX_DET, :7]


# speedup vs baseline: 6.4895x; 1.0275x over previous
import dataclasses

import jax
import jax.numpy as jnp
from jax import lax
from jax.experimental import pallas as pl
from jax.experimental.pallas import tpu as pltpu
from jax.experimental.pallas import tpu_sc as plsc

CONF = 0.3
MAX_WH = 4096.0
MAX_DET = 300
N = 20000
NCLS = 15
B = 4
D = 200
REC = 32
RECROWS = B * N * REC // 128
ROWS_PER_SUB = 1248
ROWS_MAX = 1280
GROUPS = ROWS_MAX // 16
DET_PAD = 304
NCHUNK = DET_PAD // 16
TC_R = 5120
BM_LANE = 6 + NCLS
DEG = 0.017453292519943295


def _tc_body(x_ref, rec_ref, bm_ref):
    blk = x_ref[0]
    conf = blk[4, :]
    mc = conf > CONF
    bmv = jnp.where(mc, 1 << 15, 0)
    for j in range(NCLS):
        vj = jnp.logical_and(mc, blk[5 + j, :] * conf > CONF)
        bmv = bmv + jnp.where(vj, 1 << j, 0)
    ang = blk[5 + NCLS:D, :]
    m = jnp.max(ang, axis=0, keepdims=True)
    i180 = lax.broadcasted_iota(jnp.int32, ang.shape, 0)
    aidx = jnp.min(jnp.where(ang == m, i180, jnp.int32(10**9)), axis=0)
    theta = (aidx.astype(jnp.float32) - 90.0) * jnp.float32(DEG)
    rect = jnp.concatenate([blk[0:5, :], theta[None, :], blk[5:5 + NCLS, :]],
                           axis=0)
    rec = jnp.transpose(rect)
    rec_ref[0] = jnp.concatenate(
        [rec, jnp.zeros((TC_R, 128 - 6 - NCLS), jnp.float32)], axis=1)
    bm_ref[0, 0] = bmv


@jax.jit
def _tc_prepass(xt):
    grid = (N + TC_R - 1) // TC_R
    return pl.pallas_call(
        _tc_body,
        grid=(B, grid),
        in_specs=[pl.BlockSpec((1, D, TC_R), lambda b, i: (b, 0, i))],
        out_specs=[
            pl.BlockSpec((1, TC_R, 128), lambda b, i: (b, i, 0)),
            pl.BlockSpec((1, 1, TC_R), lambda b, i: (b, 0, i)),
        ],
        out_shape=[
            jax.ShapeDtypeStruct((B, N, 128), jnp.float32),
            jax.ShapeDtypeStruct((B, 1, N), jnp.int32),
        ],
        compiler_params=pltpu.CompilerParams(
            dimension_semantics=("parallel", "parallel")),
    )(xt)


def _iota():
    return lax.iota(jnp.int32, 16)


def _splat_i32(v):
    return jnp.zeros((16,), jnp.int32) + v


def _cfun(pc_ref, bm_ref, q, t_scalar):
    lo = jnp.zeros((16,), jnp.int32)
    hi = _splat_i32(N)
    nm1 = _splat_i32(N - 1)
    for _ in range(15):
        mid = jnp.minimum(lax.shift_right_logical(lo + hi, 1), nm1)
        pv = plsc.load_gather(pc_ref, [mid])
        cond = pv > q
        hi = jnp.where(cond, mid, hi)
        lo = jnp.where(cond, lo, mid + 1)
    in_t = q < t_scalar
    r = jnp.where(in_t, lo, 0)
    pcm1 = plsc.load_gather(pc_ref, [jnp.maximum(r - 1, 0)])
    pcx = jnp.where(r > 0, pcm1, 0)
    rem = q - pcx
    bmv = plsc.load_gather(bm_ref, [r])
    col = jnp.zeros((16,), jnp.int32)
    seen = jnp.zeros((16,), jnp.int32)
    found = jnp.zeros((16,), jnp.bool_)
    for t in range(NCLS):
        bit = lax.shift_right_logical(bmv, t) & 1
        hit = (bit == 1) & (seen == rem) & jnp.logical_not(found)
        col = jnp.where(hit, t, col)
        found = jnp.logical_or(found, hit)
        seen = seen + bit
    col = jnp.where(in_t, col, 0)
    return r, col


def _sc_body(rec_ref, bm_ref, out_ref, lbm, lpc, lrank, pc_all, bm_all,
             idx3, rowbuf, outbuf, c2buf, tot_v, cnt_all,
             spm_pc, spm_bm, spm_cnt, sem):
    c = lax.axis_index("c")
    s = lax.axis_index("s")
    iv = _iota()
    rowbase = ROWS_PER_SUB * s
    nrows = jnp.where(s == 15, ROWS_MAX, ROWS_PER_SUB)

    for bb in range(2):
        b = 2 * c + bb

        pltpu.sync_copy(bm_ref.at[b].at[0].at[pl.ds(rowbase, ROWS_MAX)], lbm)

        def p1_group(g, carry):
            vcar, ccar = carry
            rid = iv + 16 * g
            rmask = rid < nrows
            bmv = lbm[pl.ds(16 * g, 16)]
            mc = (lax.shift_right_logical(bmv, 15) == 1) & rmask
            v = bmv & 0x7FFF
            v = v - (lax.shift_right_logical(v, 1) & 0x5555)
            v = (v & 0x3333) + (lax.shift_right_logical(v, 2) & 0x3333)
            v = (v + lax.shift_right_logical(v, 4)) & 0x0F0F
            cntv = (v + lax.shift_right_logical(v, 8)) & 0x1F
            cntv = jnp.where(rmask, cntv, 0)
            pcv = plsc.cumsum(cntv) + vcar
            lpc[pl.ds(16 * g, 16)] = pcv
            mci = jnp.where(mc, 1, 0)
            rkv = plsc.cumsum(mci) + ccar
            lrank[pl.ds(16 * g, 16)] = rkv
            return vcar + jnp.sum(cntv), ccar + jnp.sum(mci)

        vc_tot, cc_tot = lax.fori_loop(
            0, GROUPS, p1_group, (jnp.int32(0), jnp.int32(0)))

        totv = jnp.where(iv == 0, vc_tot, jnp.where(iv == 1, cc_tot, 0))
        tot_v[...] = totv
        pltpu.sync_copy(tot_v, spm_cnt.at[s])
        plsc.subcore_barrier()

        pltpu.sync_copy(spm_cnt, cnt_all)
        vc_col = plsc.load_gather(cnt_all, [iv, _splat_i32(0)])
        cc_col = plsc.load_gather(cnt_all, [iv, _splat_i32(1)])
        before = iv < s
        vstart = jnp.sum(jnp.where(before, vc_col, 0))
        cstart = jnp.sum(jnp.where(before, cc_col, 0))
        t_total = jnp.sum(vc_col)

        @pl.loop(0, GROUPS)
        def _adjust(g):
            sl = pl.ds(16 * g, 16)
            lpc[sl] = lpc[sl] + vstart
            lbm[sl] = (lbm[sl] & 0xFFFF) + lax.shift_left(
                lrank[sl] + cstart, 16)

        pltpu.sync_copy(lpc.at[pl.ds(0, ROWS_PER_SUB)],
                        spm_pc.at[pl.ds(rowbase, ROWS_PER_SUB)])
        pltpu.sync_copy(lbm.at[pl.ds(0, ROWS_PER_SUB)],
                        spm_bm.at[pl.ds(rowbase, ROWS_PER_SUB)])

        @pl.when(s == 15)
        def _tail():
            tail = ROWS_MAX - ROWS_PER_SUB
            src = pl.ds(ROWS_PER_SUB, tail)
            dst = pl.ds(16 * ROWS_PER_SUB, tail)
            pltpu.sync_copy(lpc.at[src], spm_pc.at[dst])
            pltpu.sync_copy(lbm.at[src], spm_bm.at[dst])

        plsc.subcore_barrier()

        pltpu.sync_copy(spm_pc, pc_all)
        pltpu.sync_copy(spm_bm, bm_all)

        def do_chunk(ch):
            jv = 16 * ch + iv
            r1, _c1 = _cfun(pc_all, bm_all, jv, t_total)
            rk = lax.shift_right_logical(
                plsc.load_gather(bm_all, [r1]), 16) - 1
            tm1 = jnp.maximum(t_total - 1, 0)
            keep = jnp.clip(rk, 0, tm1)
            r2, c2 = _cfun(pc_all, bm_all, keep, t_total)
            idx3[...] = b * N + r2
            c2buf[...] = c2
            pltpu.async_copy(rec_ref.at[idx3], rowbuf, sem).wait()

            @pl.loop(0, 16)
            def _det(d):
                c2s = plsc.load_gather(c2buf, [_splat_i32(d)])
                hdr = rowbuf[d, pl.ds(0, 16)]
                clsv = rowbuf[d, pl.ds(6, 16)]
                conf_s = plsc.load_gather(
                    rowbuf, [_splat_i32(d), _splat_i32(4)])
                theta = plsc.load_gather(
                    rowbuf, [_splat_i32(d), _splat_i32(5)])
                coff = c2s.astype(jnp.float32) * MAX_WH
                score = jnp.sum(jnp.where(iv == c2s, clsv, 0.0)) * conf_s
                outv = jnp.where(
                    iv < 4, hdr + coff,
                    jnp.where(iv == 4, theta,
                              jnp.where(iv == 5, score,
                                        jnp.where(iv == 6,
                                                  c2s.astype(jnp.float32),
                                                  0.0))))
                live = jnp.where(16 * ch + d < t_total,
                                 jnp.float32(1.0), jnp.float32(0.0))
                outbuf[d, :] = outv * live

            pltpu.sync_copy(outbuf,
                            out_ref.at[b].at[pl.ds(16 * ch, 16), :])

        do_chunk(s)

        @pl.when(s < NCHUNK - 16)
        def _extra():
            do_chunk(16 + s)

        plsc.subcore_barrier()


@jax.jit
def _nms_sc(rec, bm):
    mesh = plsc.VectorSubcoreMesh(core_axis_name="c", subcore_axis_name="s")
    cp = pltpu.CompilerParams()
    fields = pltpu.CompilerParams.__dataclass_fields__
    if "needs_layout_passes" in fields:
        cp = dataclasses.replace(cp, needs_layout_passes=False)
    if "use_tc_tiling_on_sc" in fields:
        cp = dataclasses.replace(cp, use_tc_tiling_on_sc=True)
    kfn = pl.kernel(
        _sc_body,
        out_type=jax.ShapeDtypeStruct((B, DET_PAD, 16), jnp.float32),
        mesh=mesh,
        scratch_types=[
            pltpu.VMEM((ROWS_MAX,), jnp.int32),
            pltpu.VMEM((ROWS_MAX,), jnp.int32),
            pltpu.VMEM((ROWS_MAX,), jnp.int32),
            pltpu.VMEM((N,), jnp.int32),
            pltpu.VMEM((N,), jnp.int32),
            pltpu.VMEM((16,), jnp.int32),
            pltpu.VMEM((16, 128), jnp.float32),
            pltpu.VMEM((16, 16), jnp.float32),
            pltpu.VMEM((16,), jnp.int32),
            pltpu.VMEM((16,), jnp.int32),
            pltpu.VMEM((16, 16), jnp.int32),
            pltpu.VMEM_SHARED((N,), jnp.int32),
            pltpu.VMEM_SHARED((N,), jnp.int32),
            pltpu.VMEM_SHARED((16, 16), jnp.int32),
            pltpu.SemaphoreType.DMA,
        ],
        compiler_params=cp,
    )
    return kfn(rec, bm)


def kernel(x):
    xt = x.reshape(B, N, D).transpose(0, 2, 1)
    rec, bm = _tc_prepass(xt)
    outpad = _nms_sc(rec.reshape(B * N, 128), bm)
    return outpad[:, :MAX_DET, :7]


# speedup vs baseline: 6.6459x; 1.0241x over previous
import dataclasses

import jax
import jax.numpy as jnp
from jax import lax
from jax.experimental import pallas as pl
from jax.experimental.pallas import tpu as pltpu
from jax.experimental.pallas import tpu_sc as plsc

CONF = 0.3
MAX_WH = 4096.0
MAX_DET = 300
N = 20000
NCLS = 15
B = 4
D = 200
REC = 32
RECROWS = B * N * REC // 128
ROWS_PER_SUB = 1248
ROWS_MAX = 1280
GROUPS = ROWS_MAX // 16
DET_PAD = 304
NCHUNK = DET_PAD // 16
TC_R = 10240
BM_LANE = 6 + NCLS
DEG = 0.017453292519943295


def _tc_body(x_ref, rec_ref, bm_ref):
    blk = x_ref[0]
    conf = blk[4, :]
    mc = conf > CONF
    bmv = jnp.where(mc, 1 << 15, 0)
    for j in range(NCLS):
        vj = jnp.logical_and(mc, blk[5 + j, :] * conf > CONF)
        bmv = bmv + jnp.where(vj, 1 << j, 0)
    ang = blk[5 + NCLS:D, :]
    m = jnp.max(ang, axis=0, keepdims=True)
    i180 = lax.broadcasted_iota(jnp.int32, ang.shape, 0)
    aidx = jnp.min(jnp.where(ang == m, i180, jnp.int32(10**9)), axis=0)
    theta = (aidx.astype(jnp.float32) - 90.0) * jnp.float32(DEG)
    rect = jnp.concatenate([blk[0:5, :], theta[None, :], blk[5:5 + NCLS, :]],
                           axis=0)
    rec = jnp.transpose(rect)
    rec_ref[0] = jnp.concatenate(
        [rec, jnp.zeros((TC_R, 128 - 6 - NCLS), jnp.float32)], axis=1)
    bm_ref[0, 0] = bmv


@jax.jit
def _tc_prepass(xt):
    grid = (N + TC_R - 1) // TC_R
    return pl.pallas_call(
        _tc_body,
        grid=(B, grid),
        in_specs=[pl.BlockSpec((1, D, TC_R), lambda b, i: (b, 0, i))],
        out_specs=[
            pl.BlockSpec((1, TC_R, 128), lambda b, i: (b, i, 0)),
            pl.BlockSpec((1, 1, TC_R), lambda b, i: (b, 0, i)),
        ],
        out_shape=[
            jax.ShapeDtypeStruct((B, N, 128), jnp.float32),
            jax.ShapeDtypeStruct((B, 1, N), jnp.int32),
        ],
        compiler_params=pltpu.CompilerParams(
            dimension_semantics=("parallel", "parallel")),
    )(xt)


def _iota():
    return lax.iota(jnp.int32, 16)


def _splat_i32(v):
    return jnp.zeros((16,), jnp.int32) + v


def _cfun(pc_ref, bm_ref, q, t_scalar):
    lo = jnp.zeros((16,), jnp.int32)
    hi = _splat_i32(N)
    nm1 = _splat_i32(N - 1)
    for _ in range(15):
        mid = jnp.minimum(lax.shift_right_logical(lo + hi, 1), nm1)
        pv = plsc.load_gather(pc_ref, [mid])
        cond = pv > q
        hi = jnp.where(cond, mid, hi)
        lo = jnp.where(cond, lo, mid + 1)
    in_t = q < t_scalar
    r = jnp.where(in_t, lo, 0)
    pcm1 = plsc.load_gather(pc_ref, [jnp.maximum(r - 1, 0)])
    pcx = jnp.where(r > 0, pcm1, 0)
    rem = q - pcx
    bmv = plsc.load_gather(bm_ref, [r])
    col = jnp.zeros((16,), jnp.int32)
    seen = jnp.zeros((16,), jnp.int32)
    found = jnp.zeros((16,), jnp.bool_)
    for t in range(NCLS):
        bit = lax.shift_right_logical(bmv, t) & 1
        hit = (bit == 1) & (seen == rem) & jnp.logical_not(found)
        col = jnp.where(hit, t, col)
        found = jnp.logical_or(found, hit)
        seen = seen + bit
    col = jnp.where(in_t, col, 0)
    return r, col


def _sc_body(rec_ref, bm_ref, out_ref, lbm, lpc, lrank, pc_all, bm_all,
             idx3, rowbuf, outbuf, c2buf, tot_v, cnt_all,
             spm_pc, spm_bm, spm_cnt, sem):
    c = lax.axis_index("c")
    s = lax.axis_index("s")
    iv = _iota()
    rowbase = ROWS_PER_SUB * s
    nrows = jnp.where(s == 15, ROWS_MAX, ROWS_PER_SUB)

    for bb in range(2):
        b = 2 * c + bb

        pltpu.sync_copy(bm_ref.at[b].at[0].at[pl.ds(rowbase, ROWS_MAX)], lbm)

        def p1_group(g, carry):
            vcar, ccar = carry
            rid = iv + 16 * g
            rmask = rid < nrows
            bmv = lbm[pl.ds(16 * g, 16)]
            mc = (lax.shift_right_logical(bmv, 15) == 1) & rmask
            v = bmv & 0x7FFF
            v = v - (lax.shift_right_logical(v, 1) & 0x5555)
            v = (v & 0x3333) + (lax.shift_right_logical(v, 2) & 0x3333)
            v = (v + lax.shift_right_logical(v, 4)) & 0x0F0F
            cntv = (v + lax.shift_right_logical(v, 8)) & 0x1F
            cntv = jnp.where(rmask, cntv, 0)
            pcv = plsc.cumsum(cntv) + vcar
            lpc[pl.ds(16 * g, 16)] = pcv
            mci = jnp.where(mc, 1, 0)
            rkv = plsc.cumsum(mci) + ccar
            lrank[pl.ds(16 * g, 16)] = rkv
            return vcar + jnp.sum(cntv), ccar + jnp.sum(mci)

        vc_tot, cc_tot = lax.fori_loop(
            0, GROUPS, p1_group, (jnp.int32(0), jnp.int32(0)))

        totv = jnp.where(iv == 0, vc_tot, jnp.where(iv == 1, cc_tot, 0))
        tot_v[...] = totv
        pltpu.sync_copy(tot_v, spm_cnt.at[s])
        plsc.subcore_barrier()

        pltpu.sync_copy(spm_cnt, cnt_all)
        vc_col = plsc.load_gather(cnt_all, [iv, _splat_i32(0)])
        cc_col = plsc.load_gather(cnt_all, [iv, _splat_i32(1)])
        before = iv < s
        vstart = jnp.sum(jnp.where(before, vc_col, 0))
        cstart = jnp.sum(jnp.where(before, cc_col, 0))
        t_total = jnp.sum(vc_col)

        @pl.loop(0, GROUPS)
        def _adjust(g):
            sl = pl.ds(16 * g, 16)
            lpc[sl] = lpc[sl] + vstart
            lbm[sl] = (lbm[sl] & 0xFFFF) + lax.shift_left(
                lrank[sl] + cstart, 16)

        pltpu.sync_copy(lpc.at[pl.ds(0, ROWS_PER_SUB)],
                        spm_pc.at[pl.ds(rowbase, ROWS_PER_SUB)])
        pltpu.sync_copy(lbm.at[pl.ds(0, ROWS_PER_SUB)],
                        spm_bm.at[pl.ds(rowbase, ROWS_PER_SUB)])

        @pl.when(s == 15)
        def _tail():
            tail = ROWS_MAX - ROWS_PER_SUB
            src = pl.ds(ROWS_PER_SUB, tail)
            dst = pl.ds(16 * ROWS_PER_SUB, tail)
            pltpu.sync_copy(lpc.at[src], spm_pc.at[dst])
            pltpu.sync_copy(lbm.at[src], spm_bm.at[dst])

        plsc.subcore_barrier()

        pltpu.sync_copy(spm_pc, pc_all)
        pltpu.sync_copy(spm_bm, bm_all)

        def do_chunk(ch):
            jv = 16 * ch + iv
            r1, _c1 = _cfun(pc_all, bm_all, jv, t_total)
            rk = lax.shift_right_logical(
                plsc.load_gather(bm_all, [r1]), 16) - 1
            tm1 = jnp.maximum(t_total - 1, 0)
            keep = jnp.clip(rk, 0, tm1)
            r2, c2 = _cfun(pc_all, bm_all, keep, t_total)
            idx3[...] = b * N + r2
            c2buf[...] = c2
            pltpu.async_copy(rec_ref.at[idx3], rowbuf, sem).wait()

            @pl.loop(0, 16)
            def _det(d):
                c2s = plsc.load_gather(c2buf, [_splat_i32(d)])
                hdr = rowbuf[d, pl.ds(0, 16)]
                clsv = rowbuf[d, pl.ds(6, 16)]
                conf_s = plsc.load_gather(
                    rowbuf, [_splat_i32(d), _splat_i32(4)])
                theta = plsc.load_gather(
                    rowbuf, [_splat_i32(d), _splat_i32(5)])
                coff = c2s.astype(jnp.float32) * MAX_WH
                score = jnp.sum(jnp.where(iv == c2s, clsv, 0.0)) * conf_s
                outv = jnp.where(
                    iv < 4, hdr + coff,
                    jnp.where(iv == 4, theta,
                              jnp.where(iv == 5, score,
                                        jnp.where(iv == 6,
                                                  c2s.astype(jnp.float32),
                                                  0.0))))
                live = jnp.where(16 * ch + d < t_total,
                                 jnp.float32(1.0), jnp.float32(0.0))
                outbuf[d, :] = outv * live

            pltpu.sync_copy(outbuf,
                            out_ref.at[b].at[pl.ds(16 * ch, 16), :])

        do_chunk(s)

        @pl.when(s < NCHUNK - 16)
        def _extra():
            do_chunk(16 + s)

        plsc.subcore_barrier()


@jax.jit
def _nms_sc(rec, bm):
    mesh = plsc.VectorSubcoreMesh(core_axis_name="c", subcore_axis_name="s")
    cp = pltpu.CompilerParams()
    fields = pltpu.CompilerParams.__dataclass_fields__
    if "needs_layout_passes" in fields:
        cp = dataclasses.replace(cp, needs_layout_passes=False)
    if "use_tc_tiling_on_sc" in fields:
        cp = dataclasses.replace(cp, use_tc_tiling_on_sc=True)
    kfn = pl.kernel(
        _sc_body,
        out_type=jax.ShapeDtypeStruct((B, DET_PAD, 16), jnp.float32),
        mesh=mesh,
        scratch_types=[
            pltpu.VMEM((ROWS_MAX,), jnp.int32),
            pltpu.VMEM((ROWS_MAX,), jnp.int32),
            pltpu.VMEM((ROWS_MAX,), jnp.int32),
            pltpu.VMEM((N,), jnp.int32),
            pltpu.VMEM((N,), jnp.int32),
            pltpu.VMEM((16,), jnp.int32),
            pltpu.VMEM((16, 128), jnp.float32),
            pltpu.VMEM((16, 16), jnp.float32),
            pltpu.VMEM((16,), jnp.int32),
            pltpu.VMEM((16,), jnp.int32),
            pltpu.VMEM((16, 16), jnp.int32),
            pltpu.VMEM_SHARED((N,), jnp.int32),
            pltpu.VMEM_SHARED((N,), jnp.int32),
            pltpu.VMEM_SHARED((16, 16), jnp.int32),
            pltpu.SemaphoreType.DMA,
        ],
        compiler_params=cp,
    )
    return kfn(rec, bm)


def kernel(x):
    xt = x.reshape(B, N, D).transpose(0, 2, 1)
    rec, bm = _tc_prepass(xt)
    outpad = _nms_sc(rec.reshape(B * N, 128), bm)
    return outpad[:, :MAX_DET, :7]


# speedup vs baseline: 6.9267x; 1.0422x over previous
import dataclasses

import jax
import jax.numpy as jnp
from jax import lax
from jax.experimental import pallas as pl
from jax.experimental.pallas import tpu as pltpu
from jax.experimental.pallas import tpu_sc as plsc

CONF = 0.3
MAX_WH = 4096.0
MAX_DET = 300
N = 20000
NCLS = 15
B = 4
D = 200
REC = 32
RECROWS = B * N * REC // 128
ROWS_PER_SUB = 1248
ROWS_MAX = 1280
GROUPS = ROWS_MAX // 16
DET_PAD = 304
NCHUNK = DET_PAD // 16
TC_R = 10240
BM_LANE = 6 + NCLS
DEG = 0.017453292519943295


def _tc_body(x_ref, rec_ref, bm_ref):
    blk = x_ref[0]
    conf = blk[4, :]
    mc = conf > CONF
    bmv = jnp.where(mc, 1 << 15, 0)
    for j in range(NCLS):
        vj = jnp.logical_and(mc, blk[5 + j, :] * conf > CONF)
        bmv = bmv + jnp.where(vj, 1 << j, 0)
    ang = blk[5 + NCLS:D, :]
    m = jnp.max(ang, axis=0, keepdims=True)
    i180 = lax.broadcasted_iota(jnp.int32, ang.shape, 0)
    aidx = jnp.min(jnp.where(ang == m, i180, jnp.int32(10**9)), axis=0)
    theta = (aidx.astype(jnp.float32) - 90.0) * jnp.float32(DEG)
    rect = jnp.concatenate([blk[0:5, :], theta[None, :], blk[5:5 + NCLS, :]],
                           axis=0)
    rec = jnp.transpose(rect)
    rec_ref[0] = jnp.concatenate(
        [rec, jnp.zeros((TC_R, 128 - 6 - NCLS), jnp.float32)], axis=1)
    bm_ref[0, 0] = bmv


def _make_prepass(b0):
    grid = (N + TC_R - 1) // TC_R
    return pl.pallas_call(
        _tc_body,
        grid=(2, grid),
        in_specs=[pl.BlockSpec((1, D, TC_R), lambda b, i: (b0 + b, 0, i))],
        out_specs=[
            pl.BlockSpec((1, TC_R, 128), lambda b, i: (b, i, 0)),
            pl.BlockSpec((1, 1, TC_R), lambda b, i: (b, 0, i)),
        ],
        out_shape=[
            jax.ShapeDtypeStruct((2, N, 128), jnp.float32),
            jax.ShapeDtypeStruct((2, 1, N), jnp.int32),
        ],
        compiler_params=pltpu.CompilerParams(
            dimension_semantics=("parallel", "parallel")),
    )


def _iota():
    return lax.iota(jnp.int32, 16)


def _splat_i32(v):
    return jnp.zeros((16,), jnp.int32) + v


def _cfun(pc_ref, bm_ref, q, t_scalar):
    lo = jnp.zeros((16,), jnp.int32)
    hi = _splat_i32(N)
    nm1 = _splat_i32(N - 1)
    for _ in range(15):
        mid = jnp.minimum(lax.shift_right_logical(lo + hi, 1), nm1)
        pv = plsc.load_gather(pc_ref, [mid])
        cond = pv > q
        hi = jnp.where(cond, mid, hi)
        lo = jnp.where(cond, lo, mid + 1)
    in_t = q < t_scalar
    r = jnp.where(in_t, lo, 0)
    pcm1 = plsc.load_gather(pc_ref, [jnp.maximum(r - 1, 0)])
    pcx = jnp.where(r > 0, pcm1, 0)
    rem = q - pcx
    bmv = plsc.load_gather(bm_ref, [r])
    col = jnp.zeros((16,), jnp.int32)
    seen = jnp.zeros((16,), jnp.int32)
    found = jnp.zeros((16,), jnp.bool_)
    for t in range(NCLS):
        bit = lax.shift_right_logical(bmv, t) & 1
        hit = (bit == 1) & (seen == rem) & jnp.logical_not(found)
        col = jnp.where(hit, t, col)
        found = jnp.logical_or(found, hit)
        seen = seen + bit
    col = jnp.where(in_t, col, 0)
    return r, col


def _sc_body(rec_ref, bm_ref, out_ref, lbm, lpc, lrank, pc_all, bm_all,
             idx3, rowbuf, outbuf, c2buf, tot_v, cnt_all,
             spm_pc, spm_bm, spm_cnt, sem):
    c = lax.axis_index("c")
    s = lax.axis_index("s")
    iv = _iota()
    rowbase = ROWS_PER_SUB * s
    nrows = jnp.where(s == 15, ROWS_MAX, ROWS_PER_SUB)

    if True:
        b = c

        pltpu.sync_copy(bm_ref.at[b].at[0].at[pl.ds(rowbase, ROWS_MAX)], lbm)

        def p1_group(g, carry):
            vcar, ccar = carry
            rid = iv + 16 * g
            rmask = rid < nrows
            bmv = lbm[pl.ds(16 * g, 16)]
            mc = (lax.shift_right_logical(bmv, 15) == 1) & rmask
            v = bmv & 0x7FFF
            v = v - (lax.shift_right_logical(v, 1) & 0x5555)
            v = (v & 0x3333) + (lax.shift_right_logical(v, 2) & 0x3333)
            v = (v + lax.shift_right_logical(v, 4)) & 0x0F0F
            cntv = (v + lax.shift_right_logical(v, 8)) & 0x1F
            cntv = jnp.where(rmask, cntv, 0)
            pcv = plsc.cumsum(cntv) + vcar
            lpc[pl.ds(16 * g, 16)] = pcv
            mci = jnp.where(mc, 1, 0)
            rkv = plsc.cumsum(mci) + ccar
            lrank[pl.ds(16 * g, 16)] = rkv
            return vcar + jnp.sum(cntv), ccar + jnp.sum(mci)

        vc_tot, cc_tot = lax.fori_loop(
            0, GROUPS, p1_group, (jnp.int32(0), jnp.int32(0)))

        totv = jnp.where(iv == 0, vc_tot, jnp.where(iv == 1, cc_tot, 0))
        tot_v[...] = totv
        pltpu.sync_copy(tot_v, spm_cnt.at[s])
        plsc.subcore_barrier()

        pltpu.sync_copy(spm_cnt, cnt_all)
        vc_col = plsc.load_gather(cnt_all, [iv, _splat_i32(0)])
        cc_col = plsc.load_gather(cnt_all, [iv, _splat_i32(1)])
        before = iv < s
        vstart = jnp.sum(jnp.where(before, vc_col, 0))
        cstart = jnp.sum(jnp.where(before, cc_col, 0))
        t_total = jnp.sum(vc_col)

        @pl.loop(0, GROUPS)
        def _adjust(g):
            sl = pl.ds(16 * g, 16)
            lpc[sl] = lpc[sl] + vstart
            lbm[sl] = (lbm[sl] & 0xFFFF) + lax.shift_left(
                lrank[sl] + cstart, 16)

        pltpu.sync_copy(lpc.at[pl.ds(0, ROWS_PER_SUB)],
                        spm_pc.at[pl.ds(rowbase, ROWS_PER_SUB)])
        pltpu.sync_copy(lbm.at[pl.ds(0, ROWS_PER_SUB)],
                        spm_bm.at[pl.ds(rowbase, ROWS_PER_SUB)])

        @pl.when(s == 15)
        def _tail():
            tail = ROWS_MAX - ROWS_PER_SUB
            src = pl.ds(ROWS_PER_SUB, tail)
            dst = pl.ds(16 * ROWS_PER_SUB, tail)
            pltpu.sync_copy(lpc.at[src], spm_pc.at[dst])
            pltpu.sync_copy(lbm.at[src], spm_bm.at[dst])

        plsc.subcore_barrier()

        pltpu.sync_copy(spm_pc, pc_all)
        pltpu.sync_copy(spm_bm, bm_all)

        def do_chunk(ch):
            jv = 16 * ch + iv
            r1, _c1 = _cfun(pc_all, bm_all, jv, t_total)
            rk = lax.shift_right_logical(
                plsc.load_gather(bm_all, [r1]), 16) - 1
            tm1 = jnp.maximum(t_total - 1, 0)
            keep = jnp.clip(rk, 0, tm1)
            r2, c2 = _cfun(pc_all, bm_all, keep, t_total)
            idx3[...] = b * N + r2
            c2buf[...] = c2
            pltpu.async_copy(rec_ref.at[idx3], rowbuf, sem).wait()

            @pl.loop(0, 16)
            def _det(d):
                c2s = plsc.load_gather(c2buf, [_splat_i32(d)])
                hdr = rowbuf[d, pl.ds(0, 16)]
                clsv = rowbuf[d, pl.ds(6, 16)]
                conf_s = plsc.load_gather(
                    rowbuf, [_splat_i32(d), _splat_i32(4)])
                theta = plsc.load_gather(
                    rowbuf, [_splat_i32(d), _splat_i32(5)])
                coff = c2s.astype(jnp.float32) * MAX_WH
                score = jnp.sum(jnp.where(iv == c2s, clsv, 0.0)) * conf_s
                outv = jnp.where(
                    iv < 4, hdr + coff,
                    jnp.where(iv == 4, theta,
                              jnp.where(iv == 5, score,
                                        jnp.where(iv == 6,
                                                  c2s.astype(jnp.float32),
                                                  0.0))))
                live = jnp.where(16 * ch + d < t_total,
                                 jnp.float32(1.0), jnp.float32(0.0))
                outbuf[d, :] = outv * live

            pltpu.sync_copy(outbuf,
                            out_ref.at[b].at[pl.ds(16 * ch, 16), :])

        do_chunk(s)

        @pl.when(s < NCHUNK - 16)
        def _extra():
            do_chunk(16 + s)

        plsc.subcore_barrier()


@jax.jit
def _nms_sc(rec, bm):
    mesh = plsc.VectorSubcoreMesh(core_axis_name="c", subcore_axis_name="s")
    cp = pltpu.CompilerParams()
    fields = pltpu.CompilerParams.__dataclass_fields__
    if "needs_layout_passes" in fields:
        cp = dataclasses.replace(cp, needs_layout_passes=False)
    if "use_tc_tiling_on_sc" in fields:
        cp = dataclasses.replace(cp, use_tc_tiling_on_sc=True)
    kfn = pl.kernel(
        _sc_body,
        out_type=jax.ShapeDtypeStruct((2, DET_PAD, 16), jnp.float32),
        mesh=mesh,
        scratch_types=[
            pltpu.VMEM((ROWS_MAX,), jnp.int32),
            pltpu.VMEM((ROWS_MAX,), jnp.int32),
            pltpu.VMEM((ROWS_MAX,), jnp.int32),
            pltpu.VMEM((N,), jnp.int32),
            pltpu.VMEM((N,), jnp.int32),
            pltpu.VMEM((16,), jnp.int32),
            pltpu.VMEM((16, 128), jnp.float32),
            pltpu.VMEM((16, 16), jnp.float32),
            pltpu.VMEM((16,), jnp.int32),
            pltpu.VMEM((16,), jnp.int32),
            pltpu.VMEM((16, 16), jnp.int32),
            pltpu.VMEM_SHARED((N,), jnp.int32),
            pltpu.VMEM_SHARED((N,), jnp.int32),
            pltpu.VMEM_SHARED((16, 16), jnp.int32),
            pltpu.SemaphoreType.DMA,
        ],
        compiler_params=cp,
    )
    return kfn(rec, bm)


@jax.jit
def _run(xt):
    rec_a, bm_a = _make_prepass(0)(xt)
    rec_b, bm_b = _make_prepass(2)(xt)
    out_a = _nms_sc(rec_a.reshape(2 * N, 128), bm_a)
    out_b = _nms_sc(rec_b.reshape(2 * N, 128), bm_b)
    return jnp.concatenate([out_a, out_b], axis=0)


def kernel(x):
    xt = x.reshape(B, N, D).transpose(0, 2, 1)
    outpad = _run(xt)
    return outpad[:, :MAX_DET, :7]
